# Initial kernel scaffold; baseline (speedup 1.0000x reference)
import jax
import jax.numpy as jnp
from jax.experimental import pallas as pl


def kernel(term_ids, table):
    raise NotImplementedError("write your pallas kernel here")


# speedup vs baseline: 4.4859x; 4.4859x over previous
import functools

import jax
import jax.numpy as jnp
from jax import lax
from jax.experimental import pallas as pl
from jax.experimental.pallas import tpu as pltpu
from jax.experimental.pallas import tpu_sc as plsc

N_TERMS = 100000
D = 64
B = 16384
L = 50
N_ROWS = B * L


_NORM_BLK = 2000


def _norm_body(x_ref, o_ref):
    x = x_ref[...]
    ss = jnp.sum(x * x, axis=1, keepdims=True)
    o_ref[...] = x * lax.rsqrt(jnp.maximum(ss, 1e-24))


def _normalize_table(table):
    return pl.pallas_call(
        _norm_body,
        grid=(N_TERMS // _NORM_BLK,),
        in_specs=[pl.BlockSpec((_NORM_BLK, D), lambda i: (i, 0))],
        out_specs=pl.BlockSpec((_NORM_BLK, D), lambda i: (i, 0)),
        out_shape=jax.ShapeDtypeStruct((N_TERMS, D), jnp.float32),
    )(table)


_NC = 2
_NS = 16
_NW = _NC * _NS
_PER_W = N_ROWS // _NW
_C = 128
_NCH = _PER_W // _C

_sc_mesh = plsc.VectorSubcoreMesh(core_axis_name="c", subcore_axis_name="s")


@functools.partial(
    pl.kernel,
    out_type=jax.ShapeDtypeStruct((N_ROWS, D), jnp.float32),
    mesh=_sc_mesh,
    compiler_params=pltpu.CompilerParams(use_tc_tiling_on_sc=False),
    scratch_types=[
        pltpu.VMEM((_C,), jnp.int32),
        pltpu.VMEM((_C, D), jnp.float32),
        pltpu.SemaphoreType.DMA,
    ],
)
def _gather_kernel(idx_hbm, tab_hbm, out_hbm, idx_v, rows_v, sem):
    wid = lax.axis_index("s") * _NC + lax.axis_index("c")
    base = wid * _PER_W

    def body(g, carry):
        off = base + g * _C
        pltpu.sync_copy(idx_hbm.at[pl.ds(off, _C)], idx_v)
        pltpu.async_copy(tab_hbm.at[idx_v], rows_v, sem).wait()
        pltpu.sync_copy(rows_v, out_hbm.at[pl.ds(off, _C)])
        return carry

    lax.fori_loop(0, _NCH, body, 0)


def kernel(term_ids, table):
    norm_tab = _normalize_table(table)
    flat_ids = term_ids.reshape(N_ROWS)
    out = _gather_kernel(flat_ids, norm_tab)
    return out.reshape(B, L, D)


# speedup vs baseline: 5.8017x; 1.2933x over previous
import functools

import jax
import jax.numpy as jnp
from jax import lax
from jax.experimental import pallas as pl
from jax.experimental.pallas import tpu as pltpu
from jax.experimental.pallas import tpu_sc as plsc

N_TERMS = 100000
D = 64
B = 16384
L = 50
N_ROWS = B * L


_NORM_BLK = 2000


def _norm_body(x_ref, o_ref):
    x = x_ref[...]
    ss = jnp.sum(x * x, axis=1, keepdims=True)
    o_ref[...] = x * lax.rsqrt(jnp.maximum(ss, 1e-24))


def _normalize_table(table):
    return pl.pallas_call(
        _norm_body,
        grid=(N_TERMS // _NORM_BLK,),
        in_specs=[pl.BlockSpec((_NORM_BLK, D), lambda i: (i, 0))],
        out_specs=pl.BlockSpec((_NORM_BLK, D), lambda i: (i, 0)),
        out_shape=jax.ShapeDtypeStruct((N_TERMS, D), jnp.float32),
    )(table)


_NC = 2
_NS = 16
_NW = _NC * _NS
_PER_W = N_ROWS // _NW
_C = 512
_K = _C // 128
_NCH = _PER_W // _C
_IDXROWS = _PER_W // 128

_sc_mesh = plsc.VectorSubcoreMesh(core_axis_name="c", subcore_axis_name="s")


@functools.partial(
    pl.kernel,
    out_type=jax.ShapeDtypeStruct((N_ROWS, D), jnp.float32),
    mesh=_sc_mesh,
    compiler_params=pltpu.CompilerParams(use_tc_tiling_on_sc=False),
    scratch_types=[
        pltpu.VMEM((_IDXROWS, 128), jnp.int32),
        pltpu.VMEM((2, _C, D), jnp.float32),
        pltpu.SemaphoreType.DMA,
        pltpu.SemaphoreType.DMA,
        pltpu.SemaphoreType.DMA,
        pltpu.SemaphoreType.DMA,
    ],
)
def _gather_kernel(idx_hbm, tab_hbm, out_hbm, idx_v, rows_v,
                   gsem0, gsem1, ssem0, ssem1):
    wid = lax.axis_index("s") * _NC + lax.axis_index("c")
    base = wid * _PER_W

    pltpu.sync_copy(idx_hbm.at[pl.ds(wid * _IDXROWS, _IDXROWS)], idx_v)

    gsems = (gsem0, gsem1)
    ssems = (ssem0, ssem1)

    def gather_descs(g, b):
        return [
            pltpu.make_async_copy(
                tab_hbm.at[idx_v.at[g * _K + j]],
                rows_v.at[b, pl.ds(j * 128, 128)],
                gsems[b],
            )
            for j in range(_K)
        ]

    def start_gather(g, b):
        for d_ in gather_descs(g, b):
            d_.start()

    def wait_gather(g, b):
        for d_ in gather_descs(g, b):
            d_.wait()

    def store_desc(g, b):
        return pltpu.make_async_copy(
            rows_v.at[b], out_hbm.at[pl.ds(base + g * _C, _C)], ssems[b])

    start_gather(0, 0)
    wait_gather(0, 0)
    store_desc(0, 0).start()
    start_gather(1, 1)

    def step(g, b):
        nb = 1 - b
        wait_gather(g, b)
        store_desc(g, b).start()
        store_desc(g - 1, nb).wait()
        start_gather(g + 1, nb)

    def pair(p, carry):
        step(2 * p + 1, 1)
        step(2 * p + 2, 0)
        return carry

    lax.fori_loop(0, (_NCH - 2) // 2, pair, 0)

    g_last = _NCH - 1
    wait_gather(g_last, 1)
    store_desc(g_last, 1).start()
    store_desc(g_last - 1, 0).wait()
    store_desc(g_last, 1).wait()


def kernel(term_ids, table):
    norm_tab = _normalize_table(table)
    idx2d = term_ids.reshape(N_ROWS // 128, 128)
    out = _gather_kernel(idx2d, norm_tab)
    return out.reshape(B, L, D)
